# baseline (speedup 1.0000x reference)
import functools

import jax
import jax.numpy as jnp
from jax import lax
from jax.experimental import pallas as pl
from jax.experimental.pallas import tpu as pltpu, tpu_sc as plsc

BATCH = 4096
HIST = 50
EMBED_DIM = 32
N_ROWS = BATCH * HIST

_INFO = plsc.get_sparse_core_info()
NUM_CORES = _INFO.num_cores
NUM_SUBCORES = _INFO.num_subcores
NUM_WORKERS = NUM_CORES * NUM_SUBCORES

ROWS_PER_WORKER = N_ROWS // NUM_WORKERS
STREAM_ROWS = 128
STREAMS_PER_WORKER = ROWS_PER_WORKER // STREAM_ROWS
GROUP = 10
NUM_GROUPS = STREAMS_PER_WORKER // GROUP
GROUP_ROWS = GROUP * STREAM_ROWS


def _gather_body(idx_hbm, table_hbm, out_hbm, idx_v, rows_v, sem):
    wid = lax.axis_index("s") * NUM_CORES + lax.axis_index("c")
    pltpu.sync_copy(idx_hbm.at[wid], idx_v)
    out_base = wid * ROWS_PER_WORKER

    def group(g, carry):
        copies = []
        for b in range(GROUP):
            copies.append(pltpu.async_copy(
                table_hbm.at[idx_v.at[g * GROUP + b]],
                rows_v.at[pl.ds(b * STREAM_ROWS, STREAM_ROWS)],
                sem))
        for c in copies:
            c.wait()
        pltpu.sync_copy(
            rows_v,
            out_hbm.at[pl.ds(out_base + g * GROUP_ROWS, GROUP_ROWS)])
        return carry

    lax.fori_loop(0, NUM_GROUPS, group, 0)


@functools.partial(jax.jit, static_argnames=())
def _sc_gather(idx2d, table):
    fn = functools.partial(
        pl.kernel,
        out_type=jax.ShapeDtypeStruct((N_ROWS, EMBED_DIM), jnp.float32),
        mesh=plsc.VectorSubcoreMesh(core_axis_name="c", subcore_axis_name="s"),
        scratch_types=[
            pltpu.VMEM((STREAMS_PER_WORKER, STREAM_ROWS), jnp.int32),
            pltpu.VMEM((GROUP_ROWS, EMBED_DIM), jnp.float32),
            pltpu.SemaphoreType.DMA,
        ],
        compiler_params=pltpu.CompilerParams(use_tc_tiling_on_sc=False),
    )(_gather_body)
    return fn(idx2d, table)


def kernel(event_inputs, event_table):
    idx3d = event_inputs.astype(jnp.int32).reshape(
        NUM_WORKERS, STREAMS_PER_WORKER, STREAM_ROWS)
    out = _sc_gather(idx3d, event_table)
    return out.reshape(BATCH, HIST, EMBED_DIM)


# speedup vs baseline: 1.0043x; 1.0043x over previous
import functools

import jax
import jax.numpy as jnp
from jax import lax
from jax.experimental import pallas as pl
from jax.experimental.pallas import tpu as pltpu, tpu_sc as plsc

BATCH = 4096
HIST = 50
EMBED_DIM = 32
N_ROWS = BATCH * HIST

_INFO = plsc.get_sparse_core_info()
NUM_CORES = _INFO.num_cores
NUM_SUBCORES = _INFO.num_subcores
NUM_WORKERS = NUM_CORES * NUM_SUBCORES

ROWS_PER_WORKER = N_ROWS // NUM_WORKERS
CHUNK = 1600
NUM_CHUNKS = ROWS_PER_WORKER // CHUNK


def _gather_body(idx_hbm, table_hbm, out_hbm, idx_v, buf0, buf1, sem_g,
                 sem_w):
    wid = lax.axis_index("s") * NUM_CORES + lax.axis_index("c")
    pltpu.sync_copy(idx_hbm.at[wid], idx_v)
    out_base = wid * ROWS_PER_WORKER
    bufs = (buf0, buf1)

    def gather(c):
        return pltpu.async_copy(
            table_hbm.at[idx_v.at[pl.ds(c * CHUNK, CHUNK)]],
            bufs[c % 2], sem_g)

    def writeback(c):
        return pltpu.async_copy(
            bufs[c % 2], out_hbm.at[pl.ds(out_base + c * CHUNK, CHUNK)],
            sem_w)

    g = [None] * NUM_CHUNKS
    w = [None] * NUM_CHUNKS
    g[0] = gather(0)
    for c in range(NUM_CHUNKS):
        if c + 1 < NUM_CHUNKS:
            if c >= 1:
                w[c - 1].wait()
            g[c + 1] = gather(c + 1)
        g[c].wait()
        w[c] = writeback(c)
    w[NUM_CHUNKS - 2].wait()
    w[NUM_CHUNKS - 1].wait()


@jax.jit
def _sc_gather(idx2d, table):
    fn = functools.partial(
        pl.kernel,
        out_type=jax.ShapeDtypeStruct((N_ROWS, EMBED_DIM), jnp.float32),
        mesh=plsc.VectorSubcoreMesh(core_axis_name="c", subcore_axis_name="s"),
        scratch_types=[
            pltpu.VMEM((ROWS_PER_WORKER,), jnp.int32),
            pltpu.VMEM((CHUNK, EMBED_DIM), jnp.float32),
            pltpu.VMEM((CHUNK, EMBED_DIM), jnp.float32),
            pltpu.SemaphoreType.DMA,
            pltpu.SemaphoreType.DMA,
        ],
        compiler_params=pltpu.CompilerParams(use_tc_tiling_on_sc=False),
    )(_gather_body)
    return fn(idx2d, table)


def kernel(event_inputs, event_table):
    idx2d = event_inputs.astype(jnp.int32).reshape(
        NUM_WORKERS, ROWS_PER_WORKER)
    out = _sc_gather(idx2d, event_table)
    return out.reshape(BATCH, HIST, EMBED_DIM)


# speedup vs baseline: 1.2296x; 1.2243x over previous
import functools

import jax
import jax.numpy as jnp
from jax import lax
from jax.experimental import pallas as pl
from jax.experimental.pallas import tpu as pltpu, tpu_sc as plsc

BATCH = 4096
HIST = 50
EMBED_DIM = 32
N_ROWS = BATCH * HIST

_INFO = plsc.get_sparse_core_info()
NUM_CORES = _INFO.num_cores
NUM_SUBCORES = _INFO.num_subcores
NUM_WORKERS = NUM_CORES * NUM_SUBCORES

ROWS_PER_WORKER = N_ROWS // NUM_WORKERS
CHUNK = 1600
NUM_CHUNKS = ROWS_PER_WORKER // CHUNK


def _gather_body(idx_hbm, table_hbm, out_hbm, idx_v, buf0, buf1, sem_g,
                 sem_w):
    wid = lax.axis_index("s") * NUM_CORES + lax.axis_index("c")
    pltpu.sync_copy(idx_hbm.at[wid], idx_v)
    out_base = wid * ROWS_PER_WORKER
    bufs = (buf0, buf1)

    def gather(c):
        return pltpu.async_copy(
            table_hbm.at[idx_v.at[pl.ds(c * CHUNK, CHUNK)]],
            bufs[c % 2], sem_g)

    def writeback(c):
        b0 = (out_base + c * CHUNK) // HIST
        handles = []
        for k in range(CHUNK // HIST):
            handles.append(pltpu.async_copy(
                bufs[c % 2].at[pl.ds(k * HIST, HIST)],
                out_hbm.at[b0 + k], sem_w))
        return handles

    g = [None] * NUM_CHUNKS
    w = [None] * NUM_CHUNKS
    g[0] = gather(0)
    for c in range(NUM_CHUNKS):
        if c + 1 < NUM_CHUNKS:
            if c >= 1:
                for h in w[c - 1]:
                    h.wait()
            g[c + 1] = gather(c + 1)
        g[c].wait()
        w[c] = writeback(c)
    for h in w[NUM_CHUNKS - 2]:
        h.wait()
    for h in w[NUM_CHUNKS - 1]:
        h.wait()


@jax.jit
def _sc_gather(idx2d, table):
    fn = functools.partial(
        pl.kernel,
        out_type=jax.ShapeDtypeStruct((BATCH, HIST, EMBED_DIM), jnp.float32),
        mesh=plsc.VectorSubcoreMesh(core_axis_name="c", subcore_axis_name="s"),
        scratch_types=[
            pltpu.VMEM((ROWS_PER_WORKER,), jnp.int32),
            pltpu.VMEM((CHUNK, EMBED_DIM), jnp.float32),
            pltpu.VMEM((CHUNK, EMBED_DIM), jnp.float32),
            pltpu.SemaphoreType.DMA,
            pltpu.SemaphoreType.DMA,
        ],
        compiler_params=pltpu.CompilerParams(use_tc_tiling_on_sc=False),
    )(_gather_body)
    return fn(idx2d, table)


def kernel(event_inputs, event_table):
    idx2d = event_inputs.astype(jnp.int32).reshape(
        NUM_WORKERS, ROWS_PER_WORKER)
    return _sc_gather(idx2d, event_table)
